# baseline (speedup 1.0000x reference)
import functools

import jax
import jax.numpy as jnp
from jax import lax
from jax.experimental import pallas as pl
from jax.experimental.pallas import tpu as pltpu
from jax.experimental.pallas import tpu_sc as plsc

N_NODES = 10000
IN_FEAT = 256
INTER = 512
OUT_FEAT = 256
NCHUNK = 4
CF = INTER // NCHUNK
NV = CF // 16
NTILES = 32
EB = 48
NBLK0 = 64
NBLK1 = 152
EPT0 = NBLK0 * EB
EPT1 = NBLK1 * EB
EPTMAX = max(EPT0, EPT1)
NBLOCKS = 16 * (NBLK0 + NBLK1)
EPAD = NBLOCKS * EB
IDXPAD = EPAD + EPTMAX
EWB = 256
FROWS = 632
NOUT = 16 * FROWS
FPIECES = (EB,) * 13 + (FROWS - 13 * EB,)
BN = 1000


def _stage0_body(h_ref, w_ref, b_ref, out_ref):
    a = jnp.dot(h_ref[...], w_ref[...], preferred_element_type=jnp.float32)
    a = a + b_ref[...]
    for c in range(NCHUNK):
        out_ref[c] = a[:, c * CF:(c + 1) * CF]


def _stage0(h, w1ht, b1row):
    return pl.pallas_call(
        _stage0_body,
        grid=(N_NODES // BN,),
        in_specs=[
            pl.BlockSpec((BN, IN_FEAT), lambda i: (i, 0)),
            pl.BlockSpec((IN_FEAT, INTER), lambda i: (0, 0)),
            pl.BlockSpec((1, INTER), lambda i: (0, 0)),
        ],
        out_specs=pl.BlockSpec((NCHUNK, BN, CF), lambda i: (0, i, 0)),
        out_shape=jax.ShapeDtypeStruct((NCHUNK, N_NODES, CF), jnp.float32),
    )(h, w1ht, b1row)


def _sc_edge_body(a_hbm, src_hbm, dst_hbm, ew_hbm, w1e_hbm, out_hbm,
                  src_a, dst_a, gat0, gat1, outb0, outb1, ewb0, ewb1,
                  w1e_v, gsem, ssem, esem, acc_sh):
    gat = (gat0, gat1)
    outb = (outb0, outb1)
    ewb = (ewb0, ewb1)
    cid = lax.axis_index("c")
    sid = lax.axis_index("s")
    zvec = jnp.zeros((16,), jnp.float32)
    nblk = jnp.where(cid == 0, NBLK0, NBLK1)
    blkbase = cid * (16 * NBLK0) + sid * nblk
    ebase = blkbase * EB

    pltpu.sync_copy(src_hbm.at[pl.ds(ebase, EPTMAX)], src_a)
    pltpu.sync_copy(dst_hbm.at[pl.ds(ebase, EPTMAX)], dst_a)

    def chunk(c, _):
        a_pl = a_hbm.at[c]
        pltpu.sync_copy(w1e_hbm.at[pl.ds(c * 512, 512)], w1e_v)
        cols = [[w1e_v[pl.ds(r * CF + v * 16, 16)] for v in range(NV)]
                for r in range(3)]

        def zrow(r, _):
            for v in range(NV):
                outb0[r, pl.ds(v * 16, 16)] = zvec
            return 0
        lax.fori_loop(0, EB, zrow, 0)
        done = 0
        for n in FPIECES:
            pltpu.sync_copy(outb0.at[pl.ds(0, n)],
                            acc_sh.at[pl.ds(sid * FROWS + done, n)])
            done += n
        plsc.subcore_barrier()

        pltpu.async_copy(a_pl.at[src_a.at[pl.ds(0, EB)]], gat0, gsem)
        pltpu.async_copy(ew_hbm.at[pl.ds(blkbase * EWB, EWB)], ewb0, esem)

        def pair(p, _):
            for ph in range(2):
                b = 2 * p + ph
                gat_ph = gat[ph]
                out_ph = outb[ph]
                ew_ph = ewb[ph]
                pltpu.make_async_copy(
                    a_pl.at[pl.ds(0, EB)], gat_ph, gsem).wait()
                pltpu.make_async_copy(
                    ew_hbm.at[pl.ds(0, EWB)], ew_ph, esem).wait()

                @pl.when(b + 1 < nblk)
                def _():
                    pltpu.async_copy(
                        a_pl.at[src_a.at[pl.ds((b + 1) * EB, EB)]],
                        gat[1 - ph], gsem)
                    pltpu.async_copy(
                        ew_hbm.at[pl.ds((blkbase + b + 1) * EWB, EWB)],
                        ewb[1 - ph], esem)

                @pl.when(b >= 2)
                def _():
                    pltpu.make_async_copy(
                        a_pl.at[pl.ds(0, EB)], out_ph, ssem).wait()

                zidx = jnp.zeros((16,), jnp.int32)
                dnums = lax.GatherDimensionNumbers(
                    offset_dims=(), collapsed_slice_dims=(0,),
                    start_index_map=(0,))

                @plsc.parallel_loop(0, EB, step=1, unroll=4)
                def _edge(e):
                    gbase = jnp.bitwise_and(e, -16)
                    lane = (zidx + jnp.bitwise_and(e, 15)).reshape(16, 1)
                    bw = [lax.gather(
                              ew_ph[pl.ds(k * EB + gbase, 16)], lane,
                              dnums, (1,),
                              mode=lax.GatherScatterMode.PROMISE_IN_BOUNDS)
                          for k in range(3)]
                    for v in range(NV):
                        sl = pl.ds(v * 16, 16)
                        r = (gat_ph[e, sl] + bw[0] * cols[0][v]
                             + bw[1] * cols[1][v] + bw[2] * cols[2][v])
                        out_ph[e, sl] = jnp.maximum(r, 0.0)

                pltpu.async_copy(out_ph,
                                 acc_sh.at[dst_a.at[pl.ds(b * EB, EB)]],
                                 ssem, add=True)
            return 0
        lax.fori_loop(0, nblk // 2, pair, 0)
        for ph in range(2):
            pltpu.make_async_copy(
                a_pl.at[pl.ds(0, EB)], outb[ph], ssem).wait()
        plsc.subcore_barrier()

        oj = c * 2 + cid
        rbase = sid * FROWS
        done = 0
        for n in FPIECES:
            pltpu.sync_copy(acc_sh.at[pl.ds(rbase + done, n)],
                            gat0.at[pl.ds(0, n)])
            pltpu.sync_copy(gat0.at[pl.ds(0, n)],
                            out_hbm.at[oj, pl.ds(rbase + done, n)])
            done += n
        plsc.subcore_barrier()
        return 0
    lax.fori_loop(0, NCHUNK, chunk, 0)


@functools.partial(
    pl.kernel,
    out_type=jax.ShapeDtypeStruct((2 * NCHUNK, NOUT, CF), jnp.float32),
    mesh=plsc.VectorSubcoreMesh(core_axis_name="c", subcore_axis_name="s"),
    scratch_types=[
        pltpu.VMEM((EPTMAX,), jnp.int32),
        pltpu.VMEM((EPTMAX,), jnp.int32),
        pltpu.VMEM((EB, CF), jnp.float32),
        pltpu.VMEM((EB, CF), jnp.float32),
        pltpu.VMEM((EB, CF), jnp.float32),
        pltpu.VMEM((EB, CF), jnp.float32),
        pltpu.VMEM((EWB,), jnp.float32),
        pltpu.VMEM((EWB,), jnp.float32),
        pltpu.VMEM((512,), jnp.float32),
        pltpu.SemaphoreType.DMA,
        pltpu.SemaphoreType.DMA,
        pltpu.SemaphoreType.DMA,
        pltpu.VMEM_SHARED((NOUT, CF), jnp.float32),
    ],
)
def _sc_edge(*refs):
    _sc_edge_body(*refs)


def _stage2_body(h_ref, p_ref, w2h_ref, w2s_ref, out_ref):
    x = jnp.dot(h_ref[...], w2h_ref[...], preferred_element_type=jnp.float32)
    for j in range(2 * NCHUNK):
        x = x + jnp.dot(p_ref[j], w2s_ref[j],
                        preferred_element_type=jnp.float32)
    x = jnp.maximum(x, 0.0)
    nrm = jnp.sqrt(jnp.sum(x * x, axis=1, keepdims=True))
    out_ref[...] = x / jnp.maximum(nrm, 1e-12)


def _stage2(h, partials, w2ht, w2stack):
    return pl.pallas_call(
        _stage2_body,
        grid=(N_NODES // BN,),
        in_specs=[
            pl.BlockSpec((BN, IN_FEAT), lambda i: (i, 0)),
            pl.BlockSpec((2 * NCHUNK, BN, CF), lambda i: (0, i, 0)),
            pl.BlockSpec((IN_FEAT, OUT_FEAT), lambda i: (0, 0)),
            pl.BlockSpec((2 * NCHUNK, CF, OUT_FEAT), lambda i: (0, 0, 0)),
        ],
        out_specs=pl.BlockSpec((BN, OUT_FEAT), lambda i: (i, 0)),
        out_shape=jax.ShapeDtypeStruct((N_NODES, OUT_FEAT), jnp.float32),
    )(h, partials, w2ht, w2stack)


@jax.jit
def kernel(h, edge_index, edge_w, W1, b1, W2):
    src = edge_index[0].astype(jnp.int32)
    dst = edge_index[1].astype(jnp.int32)
    e = src.shape[0]
    pad = EPAD - e
    src_p = jnp.concatenate([src, jnp.zeros((IDXPAD - e,), jnp.int32)])
    dst_p = jnp.concatenate([dst, jnp.full((IDXPAD - e,), N_NODES, jnp.int32)])
    ew_pad = jnp.concatenate(
        [edge_w, jnp.zeros((pad, 3), jnp.float32)]).astype(jnp.float32)
    ew_blk = ew_pad.reshape(NBLOCKS, EB, 3).transpose(0, 2, 1)
    ew_p = jnp.pad(ew_blk.reshape(NBLOCKS, 3 * EB),
                   ((0, 0), (0, EWB - 3 * EB))).reshape(NBLOCKS * EWB)

    w1ht = W1[:, :IN_FEAT].T
    w1e = W1[:, IN_FEAT:]
    w1e_blocks = jnp.stack([w1e[c * CF:(c + 1) * CF, :].T
                            for c in range(NCHUNK)])
    w1e_c = jnp.pad(w1e_blocks.reshape(NCHUNK, 3 * CF),
                    ((0, 0), (0, 512 - 3 * CF))).reshape(NCHUNK * 512)

    a = _stage0(h, w1ht, b1.reshape(1, INTER))

    partials = _sc_edge(a, src_p, dst_p, ew_p, w1e_c)

    w2t = W2.T
    w2ht = w2t[:IN_FEAT]
    w2stack = jnp.stack(
        [w2t[IN_FEAT + (j // 2) * CF: IN_FEAT + (j // 2 + 1) * CF]
         for j in range(2 * NCHUNK)])

    return _stage2(h, partials, w2ht, w2stack)


# speedup vs baseline: 1.5453x; 1.5453x over previous
import functools

import jax
import jax.numpy as jnp
from jax import lax
from jax.experimental import pallas as pl
from jax.experimental.pallas import tpu as pltpu
from jax.experimental.pallas import tpu_sc as plsc

N_NODES = 10000
IN_FEAT = 256
INTER = 512
OUT_FEAT = 256
NCHUNK = 4
CF = INTER // NCHUNK
NV = CF // 16
NTILES = 32
EB = 64
NBLK0 = 80
NBLK1 = 80
EPT0 = NBLK0 * EB
EPT1 = NBLK1 * EB
EPTMAX = max(EPT0, EPT1)
NBLOCKS = 16 * (NBLK0 + NBLK1)
EPAD = NBLOCKS * EB
IDXPAD = EPAD + EPTMAX
EWB = 256
FROWS = 632
NOUT = 16 * FROWS
FPIECES = (EB,) * 9 + (FROWS - 9 * EB,)
BN = 1000


def _stage0_body(h_ref, w_ref, b_ref, out_ref):
    a = jnp.dot(h_ref[...], w_ref[...], preferred_element_type=jnp.float32)
    a = a + b_ref[...]
    for c in range(NCHUNK):
        out_ref[c] = a[:, c * CF:(c + 1) * CF]


def _stage0(h, w1ht, b1row):
    return pl.pallas_call(
        _stage0_body,
        grid=(N_NODES // BN,),
        in_specs=[
            pl.BlockSpec((BN, IN_FEAT), lambda i: (i, 0)),
            pl.BlockSpec((IN_FEAT, INTER), lambda i: (0, 0)),
            pl.BlockSpec((1, INTER), lambda i: (0, 0)),
        ],
        out_specs=pl.BlockSpec((NCHUNK, BN, CF), lambda i: (0, i, 0)),
        out_shape=jax.ShapeDtypeStruct((NCHUNK, N_NODES, CF), jnp.float32),
    )(h, w1ht, b1row)


def _sc_edge_body(a_hbm, src_hbm, dst_hbm, ew_hbm, w1e_hbm, out_hbm,
                  src_a, dst_a, gat0, gat1, outb0, outb1, ewb0, ewb1,
                  w1e_v, gsem, ssem, esem, acc_sh):
    gat = (gat0, gat1)
    outb = (outb0, outb1)
    ewb = (ewb0, ewb1)
    cid = lax.axis_index("c")
    sid = lax.axis_index("s")
    zvec = jnp.zeros((16,), jnp.float32)
    nblk = jnp.where(cid == 0, NBLK0, NBLK1)
    blkbase = cid * (16 * NBLK0) + sid * nblk
    ebase = blkbase * EB

    pltpu.sync_copy(src_hbm.at[pl.ds(ebase, EPTMAX)], src_a)
    pltpu.sync_copy(dst_hbm.at[pl.ds(ebase, EPTMAX)], dst_a)

    def chunk(c, _):
        a_pl = a_hbm.at[c]
        pltpu.sync_copy(w1e_hbm.at[pl.ds(c * 512, 512)], w1e_v)
        cols = [[w1e_v[pl.ds(r * CF + v * 16, 16)] for v in range(NV)]
                for r in range(3)]

        def zrow(r, _):
            for v in range(NV):
                outb0[r, pl.ds(v * 16, 16)] = zvec
            return 0
        lax.fori_loop(0, EB, zrow, 0)
        done = 0
        for n in FPIECES:
            pltpu.sync_copy(outb0.at[pl.ds(0, n)],
                            acc_sh.at[pl.ds(sid * FROWS + done, n)])
            done += n
        plsc.subcore_barrier()

        pltpu.async_copy(a_pl.at[src_a.at[pl.ds(0, EB)]], gat0, gsem)
        pltpu.async_copy(ew_hbm.at[pl.ds(blkbase * EWB, EWB)], ewb0, esem)

        def pair(p, _):
            for ph in range(2):
                b = 2 * p + ph
                gat_ph = gat[ph]
                out_ph = outb[ph]
                ew_ph = ewb[ph]
                pltpu.make_async_copy(
                    a_pl.at[pl.ds(0, EB)], gat_ph, gsem).wait()
                pltpu.make_async_copy(
                    ew_hbm.at[pl.ds(0, EWB)], ew_ph, esem).wait()

                @pl.when(b + 1 < nblk)
                def _():
                    pltpu.async_copy(
                        a_pl.at[src_a.at[pl.ds((b + 1) * EB, EB)]],
                        gat[1 - ph], gsem)
                    pltpu.async_copy(
                        ew_hbm.at[pl.ds((blkbase + b + 1) * EWB, EWB)],
                        ewb[1 - ph], esem)

                @pl.when(b >= 2)
                def _():
                    pltpu.make_async_copy(
                        a_pl.at[pl.ds(0, EB)], out_ph, ssem).wait()

                zidx = jnp.zeros((16,), jnp.int32)
                dnums = lax.GatherDimensionNumbers(
                    offset_dims=(), collapsed_slice_dims=(0,),
                    start_index_map=(0,))

                @plsc.parallel_loop(0, EB, step=1, unroll=4)
                def _edge(e):
                    gbase = jnp.bitwise_and(e, -16)
                    lane = (zidx + jnp.bitwise_and(e, 15)).reshape(16, 1)
                    bw = [lax.gather(
                              ew_ph[pl.ds(k * EB + gbase, 16)], lane,
                              dnums, (1,),
                              mode=lax.GatherScatterMode.PROMISE_IN_BOUNDS)
                          for k in range(3)]
                    for v in range(NV):
                        sl = pl.ds(v * 16, 16)
                        r = (gat_ph[e, sl] + bw[0] * cols[0][v]
                             + bw[1] * cols[1][v] + bw[2] * cols[2][v])
                        out_ph[e, sl] = jnp.maximum(r, 0.0)

                pltpu.async_copy(out_ph,
                                 acc_sh.at[dst_a.at[pl.ds(b * EB, EB)]],
                                 ssem, add=True)
            return 0
        lax.fori_loop(0, nblk // 2, pair, 0)
        for ph in range(2):
            pltpu.make_async_copy(
                a_pl.at[pl.ds(0, EB)], outb[ph], ssem).wait()
        plsc.subcore_barrier()

        oj = c * 2 + cid
        rbase = sid * FROWS
        done = 0
        for n in FPIECES:
            pltpu.sync_copy(acc_sh.at[pl.ds(rbase + done, n)],
                            gat0.at[pl.ds(0, n)])
            pltpu.sync_copy(gat0.at[pl.ds(0, n)],
                            out_hbm.at[oj, pl.ds(rbase + done, n)])
            done += n
        plsc.subcore_barrier()
        return 0
    lax.fori_loop(0, NCHUNK, chunk, 0)


@functools.partial(
    pl.kernel,
    out_type=jax.ShapeDtypeStruct((2 * NCHUNK, NOUT, CF), jnp.float32),
    mesh=plsc.VectorSubcoreMesh(core_axis_name="c", subcore_axis_name="s"),
    scratch_types=[
        pltpu.VMEM((EPTMAX,), jnp.int32),
        pltpu.VMEM((EPTMAX,), jnp.int32),
        pltpu.VMEM((EB, CF), jnp.float32),
        pltpu.VMEM((EB, CF), jnp.float32),
        pltpu.VMEM((EB, CF), jnp.float32),
        pltpu.VMEM((EB, CF), jnp.float32),
        pltpu.VMEM((EWB,), jnp.float32),
        pltpu.VMEM((EWB,), jnp.float32),
        pltpu.VMEM((512,), jnp.float32),
        pltpu.SemaphoreType.DMA,
        pltpu.SemaphoreType.DMA,
        pltpu.SemaphoreType.DMA,
        pltpu.VMEM_SHARED((NOUT, CF), jnp.float32),
    ],
)
def _sc_edge(*refs):
    _sc_edge_body(*refs)


def _stage2_body(h_ref, p_ref, w2h_ref, w2s_ref, out_ref):
    x = jnp.dot(h_ref[...], w2h_ref[...], preferred_element_type=jnp.float32)
    for j in range(2 * NCHUNK):
        x = x + jnp.dot(p_ref[j], w2s_ref[j],
                        preferred_element_type=jnp.float32)
    x = jnp.maximum(x, 0.0)
    nrm = jnp.sqrt(jnp.sum(x * x, axis=1, keepdims=True))
    out_ref[...] = x / jnp.maximum(nrm, 1e-12)


def _stage2(h, partials, w2ht, w2stack):
    return pl.pallas_call(
        _stage2_body,
        grid=(N_NODES // BN,),
        in_specs=[
            pl.BlockSpec((BN, IN_FEAT), lambda i: (i, 0)),
            pl.BlockSpec((2 * NCHUNK, BN, CF), lambda i: (0, i, 0)),
            pl.BlockSpec((IN_FEAT, OUT_FEAT), lambda i: (0, 0)),
            pl.BlockSpec((2 * NCHUNK, CF, OUT_FEAT), lambda i: (0, 0, 0)),
        ],
        out_specs=pl.BlockSpec((BN, OUT_FEAT), lambda i: (i, 0)),
        out_shape=jax.ShapeDtypeStruct((N_NODES, OUT_FEAT), jnp.float32),
    )(h, partials, w2ht, w2stack)


@jax.jit
def kernel(h, edge_index, edge_w, W1, b1, W2):
    src = edge_index[0].astype(jnp.int32)
    dst = edge_index[1].astype(jnp.int32)
    e = src.shape[0]
    pad = EPAD - e
    src_p = jnp.concatenate([src, jnp.zeros((IDXPAD - e,), jnp.int32)])
    dst_p = jnp.concatenate([dst, jnp.full((IDXPAD - e,), N_NODES, jnp.int32)])
    ew_pad = jnp.concatenate(
        [edge_w, jnp.zeros((pad, 3), jnp.float32)]).astype(jnp.float32)
    ew_blk = ew_pad.reshape(NBLOCKS, EB, 3).transpose(0, 2, 1)
    ew_p = jnp.pad(ew_blk.reshape(NBLOCKS, 3 * EB),
                   ((0, 0), (0, EWB - 3 * EB))).reshape(NBLOCKS * EWB)

    w1ht = W1[:, :IN_FEAT].T
    w1e = W1[:, IN_FEAT:]
    w1e_blocks = jnp.stack([w1e[c * CF:(c + 1) * CF, :].T
                            for c in range(NCHUNK)])
    w1e_c = jnp.pad(w1e_blocks.reshape(NCHUNK, 3 * CF),
                    ((0, 0), (0, 512 - 3 * CF))).reshape(NCHUNK * 512)

    a = _stage0(h, w1ht, b1.reshape(1, INTER))

    partials = _sc_edge(a, src_p, dst_p, ew_p, w1e_c)

    w2t = W2.T
    w2ht = w2t[:IN_FEAT]
    w2stack = jnp.stack(
        [w2t[IN_FEAT + (j // 2) * CF: IN_FEAT + (j // 2 + 1) * CF]
         for j in range(2 * NCHUNK)])

    return _stage2(h, partials, w2ht, w2stack)
